# baseline (speedup 1.0000x reference)
import jax
import jax.numpy as jnp
from jax import lax
from jax.experimental import pallas as pl
from jax.experimental.pallas import tpu as pltpu
from jax.experimental.pallas import tpu_sc as plsc

B = 1024
N = 2048
NA = 8943
NAPAD = 8944
VA = NAPAD // 16
VN = N // 16
NBIN = 1024
HV = NBIN // 16
MM_SEQ = 103
MM_REM = 2236
MM_ADD = 90
P_SEQ = 0.05
P_REM = 0.25
P_ADD = 0.01
NW = 32
ROWS_PER_W = B // NW
UNROLL = 4


def _mesh():
    return plsc.VectorSubcoreMesh(core_axis_name="c", subcore_axis_name="s")


def _last(v):
    return v[15]


def _body(seq_h, ann_h, rseq_h, rann_h, radd_h, rtok_h, oseq_h, oann_h,
          a2, ra2, rad2, hist, s2, rs2, rt2,
          sem_in0, sem_in1, sem_oa0, sem_oa1, sem_os0, sem_os1):
    iota = lax.iota(jnp.int32, 16)
    ones = jnp.ones((16,), jnp.int32)
    zeros = jnp.zeros((16,), jnp.int32)
    wid = lax.axis_index("s") * 2 + lax.axis_index("c")
    base = wid * ROWS_PER_W
    sem_in = (sem_in0, sem_in1)
    sem_oa = (sem_oa0, sem_oa1)
    sem_os = (sem_os0, sem_os1)
    tail_valid = ((VA - 1) * 16 + iota) < NA

    def in_copies(r, p):
        na = pl.ds(0, NA)
        return (
            pltpu.make_async_copy(ann_h.at[r], a2.at[p].at[na], sem_in[p]),
            pltpu.make_async_copy(rann_h.at[r], ra2.at[p].at[na], sem_in[p]),
            pltpu.make_async_copy(radd_h.at[r], rad2.at[p].at[na], sem_in[p]),
            pltpu.make_async_copy(seq_h.at[r], s2.at[p], sem_in[p]),
            pltpu.make_async_copy(rseq_h.at[r], rs2.at[p], sem_in[p]),
            pltpu.make_async_copy(rtok_h.at[r], rt2.at[p], sem_in[p]),
        )

    def out_copies(r, p):
        na = pl.ds(0, NA)
        return (
            pltpu.make_async_copy(a2.at[p].at[na], oann_h.at[r], sem_oa[p]),
            pltpu.make_async_copy(s2.at[p], oseq_h.at[r], sem_os[p]),
        )

    def fetch(r, p):
        for c in in_copies(r, p):
            c.start()

    def clear():
        def cb(h):
            hist[pl.ds(h * 16, 16)] = zeros
        plsc.parallel_loop(0, HV, unroll=8)(cb)

    def count_t(mask_at, mm, prod):
        nv = (mm + 15) // 16

        def tb(v, car, lv):
            cum, tacc = car
            mk = mask_at(v)
            c = plsc.cumsum(mk.astype(jnp.int32)) + cum
            ok = (c.astype(jnp.float32) - 1.0) < prod
            if lv is not None:
                ok = ok & lv
            tacc = tacc + plsc.all_reduce_population_count(ok)
            return (_last(c), tacc)

        car = plsc.parallel_loop(0, nv - 1, unroll=UNROLL,
                                 carry=(jnp.int32(0), zeros))(
            lambda v, c: tb(v, c, None))
        _, tvec = tb(nv - 1, car, ((nv - 1) * 16 + iota) < mm)
        return tvec

    def hist_scan(target):
        def hb(h, car):
            cum, rst, gvec = car
            hv = hist[pl.ds(h * 16, 16)]
            cs = plsc.cumsum(hv) + cum
            lt = cs < target
            rst = rst + plsc.all_reduce_population_count(lt)
            gvec = gvec + jnp.where(lt, hv, 0)
            return (_last(cs), rst, gvec)

        _, rst, gvec = plsc.parallel_loop(
            0, HV, unroll=UNROLL, carry=(jnp.int32(0), zeros, zeros))(hb)
        return rst, jnp.sum(gvec)

    def build_hist(key_at, nv, shift, phigh, peel_tail):
        clear()

        def bb(v, tail):
            k = key_at(v, tail)
            rb = 1023 - ((k >> shift) & 1023)
            if phigh is None:
                plsc.addupdate_scatter(hist, [rb], ones)
            else:
                pm = (k >> (shift + 10)) == phigh
                plsc.addupdate_scatter(hist, [rb], pm.astype(jnp.int32))

        plsc.parallel_loop(0, nv - 1 if peel_tail else nv, unroll=UNROLL)(
            lambda v: bb(v, False))
        if peel_tail:
            bb(nv - 1, True)

    def radix_select(key_at, nv, tvec, m_scal, l1_ready, peel_tail):
        def trivial(_):
            return zeros, tvec - m_scal

        def generic(_):
            if not l1_ready:
                build_hist(key_at, nv, 20, None, peel_tail)
            rst, gad = hist_scan(tvec)
            cbin = plsc.load_gather(hist, [rst])
            p1v = (1023 - rst) << 20
            g1 = gad
            n1 = tvec - gad

            def lvl23(_):
                build_hist(key_at, nv, 10, p1v >> 20, peel_tail)
                t2 = tvec - g1
                rst2, gad2 = hist_scan(t2)
                c2 = plsc.load_gather(hist, [rst2])
                p2 = p1v | ((1023 - rst2) << 10)
                g2 = g1 + gad2
                n2 = t2 - gad2

                def lvl3(_):
                    build_hist(key_at, nv, 0, p2 >> 10, peel_tail)
                    rst3, gad3 = hist_scan(tvec - g2)
                    p3 = p2 | (1023 - rst3)
                    return p3, tvec - g2 - gad3

                def skip3(_):
                    return p2 - 1, zeros

                return lax.cond(_last((c2 == n2).astype(jnp.int32)) == 1,
                                skip3, lvl3, 0)

            def skip23(_):
                return p1v - 1, zeros

            return lax.cond(_last((cbin == n1).astype(jnp.int32)) == 1,
                            skip23, lvl23, 0)

        return lax.cond(_last(tvec) > m_scal, trivial, generic, 0)

    def annot_row(a_buf, ra_buf, rad_buf):
        def kr_at(v, tail):
            sl = pl.ds(v * 16, 16)
            pos = a_buf[sl] > 0.0
            if tail:
                pos = pos & tail_valid
            return jnp.where(pos, plsc.bitcast(ra_buf[sl], jnp.int32) + 1, 0)

        def ka_at(v, tail):
            sl = pl.ds(v * 16, 16)
            neg = jnp.logical_not(a_buf[sl] > 0.0)
            if tail:
                neg = neg & tail_valid
            return jnp.where(neg, plsc.bitcast(rad_buf[sl], jnp.int32) + 1, 0)

        clear()

        def p1(v, mcar, tail):
            sl = pl.ds(v * 16, 16)
            pos = a_buf[sl] > 0.0
            if tail:
                pos = pos & tail_valid
            kr = jnp.where(pos, plsc.bitcast(ra_buf[sl], jnp.int32) + 1, 0)
            plsc.addupdate_scatter(hist, [1023 - (kr >> 20)], ones)
            return mcar + pos.astype(jnp.int32)

        mvec = plsc.parallel_loop(0, VA - 1, unroll=UNROLL, carry=zeros)(
            lambda v, mcar: p1(v, mcar, False))
        mvec = p1(VA - 1, mvec, True)
        m_r = jnp.sum(mvec)
        m_a = NA - m_r
        prod_r = m_r.astype(jnp.float32) * jnp.float32(P_REM)
        prod_a = m_a.astype(jnp.float32) * jnp.float32(P_ADD)

        def mask_r_at(v):
            return a_buf[pl.ds(v * 16, 16)] > 0.0

        def mask_a_at(v):
            return jnp.logical_not(a_buf[pl.ds(v * 16, 16)] > 0.0)

        t_r = count_t(mask_r_at, MM_REM, prod_r)
        t_a = count_t(mask_a_at, MM_ADD, prod_a)

        k_r, res_r = radix_select(kr_at, VA, t_r, m_r, True, True)
        k_a, res_a = radix_select(ka_at, VA, t_a, m_a, False, True)

        def make_fb(slow_r, a_mode):
            def run(_):
                def body(v, car):
                    cr, ca = car
                    sl = pl.ds(v * 16, 16)
                    a = a_buf[sl]
                    pos = a > 0.0
                    kr = jnp.where(
                        pos, plsc.bitcast(ra_buf[sl], jnp.int32) + 1, 0)
                    gt_r = kr > k_r
                    if slow_r:
                        eq_r = kr == k_r
                        rr = plsc.cumsum(eq_r.astype(jnp.int32)) + cr
                        sel_r = gt_r | (eq_r & (rr <= res_r))
                        cr = _last(rr)
                    else:
                        sel_r = gt_r
                    neg = jnp.logical_not(pos)
                    if a_mode == 'zero':
                        aa = plsc.cumsum(pos.astype(jnp.int32)) + ca
                        sel_a = neg | (aa <= res_a)
                        ca = _last(aa)
                    else:
                        ka = jnp.where(
                            neg, plsc.bitcast(rad_buf[sl], jnp.int32) + 1, 0)
                        if a_mode == 'gen':
                            eq_a = ka == k_a
                            aa = plsc.cumsum(eq_a.astype(jnp.int32)) + ca
                            sel_a = (ka > k_a) | (eq_a & (aa <= res_a))
                            ca = _last(aa)
                        else:
                            sel_a = ka > k_a
                    out = (a + jnp.where(sel_a, 1.0, 0.0)) * \
                        jnp.where(sel_r, 0.0, 1.0)
                    a_buf[sl] = out
                    return (cr, ca)

                plsc.parallel_loop(
                    0, VA, unroll=UNROLL,
                    carry=(jnp.int32(0), jnp.int32(0)))(body)
                return 0
            return run

        kazero = _last(k_a) == 0

        def on_rfast(_):
            return lax.cond(
                kazero, make_fb(False, 'zero'),
                lambda __: lax.cond(_last(res_a) == 0,
                                    make_fb(False, 'fast'),
                                    make_fb(False, 'gen'), 0), 0)

        def on_rslow(_):
            return lax.cond(kazero, make_fb(True, 'zero'),
                            make_fb(True, 'gen'), 0)

        lax.cond(_last(res_r) == 0, on_rfast, on_rslow, 0)

    def seq_row(s_buf, rs_buf, rt_buf):
        def ks_at(v, tail):
            sl = pl.ds(v * 16, 16)
            mk = s_buf[sl] > 2
            return jnp.where(mk, plsc.bitcast(rs_buf[sl], jnp.int32) + 1, 0)

        clear()

        def p1(v, mcar):
            sl = pl.ds(v * 16, 16)
            mk = s_buf[sl] > 2
            ks = jnp.where(mk, plsc.bitcast(rs_buf[sl], jnp.int32) + 1, 0)
            plsc.addupdate_scatter(hist, [1023 - (ks >> 20)], ones)
            return mcar + mk.astype(jnp.int32)

        mvec = plsc.parallel_loop(0, VN, unroll=UNROLL, carry=zeros)(p1)
        m_s = jnp.sum(mvec)
        prod_s = m_s.astype(jnp.float32) * jnp.float32(P_SEQ)

        def mask_s_at(v):
            return s_buf[pl.ds(v * 16, 16)] > 2

        t_s = count_t(mask_s_at, MM_SEQ, prod_s)
        k_s, res_s = radix_select(ks_at, VN, t_s, m_s, True, False)

        def fb_fast(_):
            def body(v):
                sl = pl.ds(v * 16, 16)
                mk = s_buf[sl] > 2
                ks = jnp.where(
                    mk, plsc.bitcast(rs_buf[sl], jnp.int32) + 1, 0)
                rt = rt_buf[sl]
                sel = (ks > k_s) & (rt > 2)
                s_buf[sl] = jnp.where(sel, rt, s_buf[sl])
            plsc.parallel_loop(0, VN, unroll=UNROLL)(body)
            return 0

        def fb_slow(_):
            def body(v, cs):
                sl = pl.ds(v * 16, 16)
                s = s_buf[sl]
                mk = s > 2
                ks = jnp.where(
                    mk, plsc.bitcast(rs_buf[sl], jnp.int32) + 1, 0)
                rt = rt_buf[sl]
                eq = ks == k_s
                cc = plsc.cumsum(eq.astype(jnp.int32)) + cs
                sel = (ks > k_s) | (eq & (cc <= res_s))
                sel = sel & (rt > 2)
                s_buf[sl] = jnp.where(sel, rt, s)
                return _last(cc)

            plsc.parallel_loop(0, VN, unroll=UNROLL, carry=jnp.int32(0))(body)
            return 0

        lax.cond(_last(res_s) == 0, fb_fast, fb_slow, 0)

    fetch(base, 0)

    def step(j, _):
        for ph in (0, 1):
            i = j * 2 + ph
            r = base + i
            q = 1 - ph
            for c in in_copies(r, ph):
                c.wait()
            annot_row(a2.at[ph], ra2.at[ph], rad2.at[ph])
            oc_a, oc_s = out_copies(r, ph)
            oc_a.start()

            @pl.when(i + 1 < ROWS_PER_W)
            def _():
                @pl.when(i >= 1)
                def _():
                    poa, pos_ = out_copies(r - 1, q)
                    poa.wait()
                    pos_.wait()
                fetch(r + 1, q)

            seq_row(s2.at[ph], rs2.at[ph], rt2.at[ph])
            oc_s.start()
        return 0

    lax.fori_loop(0, ROWS_PER_W // 2, step, 0)
    for ph, r in ((0, base + ROWS_PER_W - 2), (1, base + ROWS_PER_W - 1)):
        oa, os_ = out_copies(r, ph)
        oa.wait()
        os_.wait()


@jax.jit
def _impl(seq, annotation, rand_seq, rand_annot, rand_add, random_tokens):
    fn = pl.kernel(
        _body,
        out_type=(
            jax.ShapeDtypeStruct((B, N), jnp.int32),
            jax.ShapeDtypeStruct((B, NA), jnp.float32),
        ),
        mesh=_mesh(),
        compiler_params=pltpu.CompilerParams(
            needs_layout_passes=False, use_tc_tiling_on_sc=False),
        scratch_types=[
            pltpu.VMEM((2, NAPAD), jnp.float32),
            pltpu.VMEM((2, NAPAD), jnp.float32),
            pltpu.VMEM((2, NAPAD), jnp.float32),
            pltpu.VMEM((NBIN,), jnp.int32),
            pltpu.VMEM((2, N), jnp.int32),
            pltpu.VMEM((2, N), jnp.float32),
            pltpu.VMEM((2, N), jnp.int32),
            pltpu.SemaphoreType.DMA,
            pltpu.SemaphoreType.DMA,
            pltpu.SemaphoreType.DMA,
            pltpu.SemaphoreType.DMA,
            pltpu.SemaphoreType.DMA,
            pltpu.SemaphoreType.DMA,
        ],
    )
    return fn(seq, annotation, rand_seq, rand_annot, rand_add, random_tokens)


def kernel(seq, annotation, rand_seq, rand_annot, rand_batch, rand_add,
           random_tokens):
    del rand_batch
    return _impl(seq, annotation, rand_seq, rand_annot, rand_add,
                 random_tokens)
